# baseline (speedup 1.0000x reference)
import jax
import jax.numpy as jnp
from jax.experimental import pallas as pl
from jax.experimental.pallas import tpu as pltpu

_B, _T, _D, _C = 10, 2048, 2048, 20
_K = 292
_DT = 512
_ND = _D // _DT
_NPAIR = 3
_RP = 24
_INT_MIN = -(2 ** 31)
_M31 = 2 ** 31 - 1

_PASS_SCHED = {s: list(range(30 - 3 * (s - 1), 30 - 3 * s, -1))
               for s in range(1, 10)}
_PASS_SCHED[10] = [3, 2, 1, 0]


def _sortable(bits):
    return jnp.where(bits < 0, bits ^ _M31, bits)


def _mil(il, lab):
    lwb = lab * (1.0 / (jnp.sum(lab) + 1e-4))
    mx = jnp.max(il)
    ls = il - mx - jnp.log(jnp.sum(jnp.exp(il - mx)))
    return -jnp.sum(lwb * ls)


def _fused_kernel(cas_ref, atn_ref, mask_ref, v_ref, f_ref, labb_ref,
                  labs_ref, caso_ref, atno_ref, x1_ref, x2_ref,
                  scal_ref, milv_ref, pairs_ref,
                  stage_ref, valt_ref, keyt_ref, t_ref, am_ref, macc_ref):
    p = pl.program_id(0)
    d = pl.program_id(1)
    sidx = p * _ND + d
    kf = jnp.float32(_K)

    def cnt_ge(c):
        parts = []
        for j in range(16):
            blk = keyt_ref[pl.ds(128 * j, 128), :]
            parts.append(jnp.sum((blk >= c).astype(jnp.float32),
                                 axis=0, keepdims=True))
        while len(parts) > 1:
            parts = [a + b for a, b in zip(parts[0::2], parts[1::2])]
        return parts[0]

    @pl.when(sidx == 0)
    def _build():
        mutual = jnp.float32(0.0)
        norm_a = jnp.float32(0.0)
        norm_v = jnp.float32(0.0)
        norm_f = jnp.float32(0.0)
        guide_a = jnp.float32(0.0)
        guide_v = jnp.float32(0.0)
        guide_f = jnp.float32(0.0)
        inv_t = jnp.float32(1.0 / _T)
        pad = jnp.full((_RP - _C - 1, _T), -jnp.inf, jnp.float32)
        for i in range(_B):
            cas = cas_ref[i]
            atn = atn_ref[i]
            msk = mask_ref[i]
            v = v_ref[i] * msk
            f = f_ref[i] * msk
            el = cas * msk
            atn_m = atn * msk
            mutual += jnp.mean((v - f) ** 2)
            mn = jnp.min(el, axis=0, keepdims=True)
            supp = atn_m * (el - mn) + mn
            stage_ref[pl.ds(_RP * i, _RP), :] = \
                jnp.concatenate([el, pad], axis=0)
            stage_ref[pl.ds(_RP * (_B + i), _RP), :] = \
                jnp.concatenate([supp, pad], axis=0)
            mx = jnp.max(el, axis=0, keepdims=True)
            z = jnp.sum(jnp.exp(el - mx), axis=0, keepdims=True)
            bg = jnp.exp(el[_C:_C + 1, :] - mx) / z
            norm_a += jnp.sum(atn_m) * inv_t
            norm_v += jnp.sum(v) * inv_t
            norm_f += jnp.sum(f) * inv_t
            guide_a += jnp.sum(jnp.abs(1.0 - atn_m - bg)) * inv_t
            guide_v += jnp.sum(jnp.abs(1.0 - v - bg)) * inv_t
            guide_f += jnp.sum(jnp.abs(1.0 - f - bg)) * inv_t
        ipad = jnp.full((8, _T), -jnp.inf, jnp.float32)
        for g in range(4):
            blk = jnp.concatenate(
                [stage_ref[pl.ds(120 * g, 120), :], ipad], axis=0)
            valt_ref[:, 128 * g:128 * (g + 1)] = jnp.transpose(blk, (1, 0))
        keyt_ref[...] = _sortable(
            jax.lax.bitcast_convert_type(valt_ref[...], jnp.int32))
        zero = jnp.zeros((1, 512), jnp.int32)
        t_ref[...] = jnp.where(cnt_ge(zero) >= kf, zero,
                               jnp.full((1, 512), _INT_MIN, jnp.int32))
        scal_ref[...] = jnp.concatenate(
            [jnp.reshape(s, (1, 1)) for s in
             (mutual, norm_a, norm_v, norm_f, guide_a, guide_v, guide_f)]
            + [jnp.zeros((1, 9), jnp.float32)], axis=1)

    for s, bits in _PASS_SCHED.items():
        @pl.when(sidx == s)
        def _passes(bits=bits):
            for b in bits:
                t = t_ref[...]
                cand = t + jnp.int32(1 << b)
                t_ref[...] = jnp.where(cnt_ge(cand) >= kf, cand, t)

    @pl.when(d == 0)
    def _amats():
        for q in range(2):
            idx = 2 * p + q
            cas = caso_ref[idx]
            atn = atno_ref[idx]
            mnu = jnp.min(cas, axis=1, keepdims=True)
            st = atn * (cas - mnu) + mnu
            mxt = jnp.max(st, axis=0, keepdims=True)
            e = jnp.exp(st - mxt)
            zt = jnp.sum(e, axis=0, keepdims=True)
            a1 = e / zt
            al = (1.0 - a1) * jnp.float32(1.0 / (_T - 1))
            zpad = jnp.zeros((_T, 3), jnp.float32)
            am_ref[q] = jnp.concatenate([a1, zpad, al, zpad], axis=1)

    dn = (((0,), (0,)), ((), ()))
    a1c = am_ref[0, pl.ds(_DT * d, _DT), :]
    a2c = am_ref[1, pl.ds(_DT * d, _DT), :]
    m1 = jax.lax.dot_general(a1c, x1_ref[0], dn,
                             preferred_element_type=jnp.float32,
                             precision=jax.lax.Precision.DEFAULT)
    m2 = jax.lax.dot_general(a2c, x2_ref[0], dn,
                             preferred_element_type=jnp.float32,
                             precision=jax.lax.Precision.DEFAULT)

    @pl.when(d == 0)
    def _mzero():
        macc_ref[0] = m1
        macc_ref[1] = m2

    @pl.when(d > 0)
    def _macc():
        macc_ref[0] = macc_ref[0] + m1
        macc_ref[1] = macc_ref[1] + m2

    @pl.when(d == _ND - 1)
    def _pair_final():
        h1 = macc_ref[0][0:_C + 1]
        l1 = macc_ref[0][24:24 + _C + 1]
        h2 = macc_ref[1][0:_C + 1]
        l2 = macc_ref[1][24:24 + _C + 1]
        h1h2 = jnp.sum(h1 * h2, axis=1, keepdims=True)
        h1l2 = jnp.sum(h1 * l2, axis=1, keepdims=True)
        h2l1 = jnp.sum(h2 * l1, axis=1, keepdims=True)
        nh1 = jnp.sqrt(jnp.sum(h1 * h1, axis=1, keepdims=True))
        nh2 = jnp.sqrt(jnp.sum(h2 * h2, axis=1, keepdims=True))
        nl1 = jnp.sqrt(jnp.sum(l1 * l1, axis=1, keepdims=True))
        nl2 = jnp.sqrt(jnp.sum(l2 * l2, axis=1, keepdims=True))
        d1 = 1.0 - h1h2 / (nh1 * nh2)
        d2 = 1.0 - h1l2 / (nh1 * nl2)
        d3 = 1.0 - h2l1 / (nh2 * nl1)
        ll = labs_ref[2 * p] * labs_ref[2 * p + 1]
        part = 0.5 * (jnp.sum(jnp.maximum(d1 - d2 + 0.5, 0.0) * ll)
                      + jnp.sum(jnp.maximum(d1 - d3 + 0.5, 0.0) * ll))
        ntmp = jnp.sum(ll)
        pairs_ref[0] = jnp.concatenate(
            [jnp.reshape(part, (1, 1)), jnp.reshape(ntmp, (1, 1))], axis=1)

    @pl.when(sidx == _NPAIR * _ND - 1)
    def _finalize():
        t = t_ref[...]
        cparts = []
        sparts = []
        for j in range(16):
            kblk = keyt_ref[pl.ds(128 * j, 128), :]
            vblk = valt_ref[pl.ds(128 * j, 128), :]
            gtb = kblk > t
            cparts.append(jnp.sum(gtb.astype(jnp.float32),
                                  axis=0, keepdims=True))
            sparts.append(jnp.sum(jnp.where(gtb, vblk, jnp.float32(0.0)),
                                  axis=0, keepdims=True))
        while len(cparts) > 1:
            cparts = [a + b for a, b in zip(cparts[0::2], cparts[1::2])]
            sparts = [a + b for a, b in zip(sparts[0::2], sparts[1::2])]
        tval = jax.lax.bitcast_convert_type(_sortable(t), jnp.float32)
        il_vec = (sparts[0] + (kf - cparts[0]) * tval) \
            * jnp.float32(1.0 / _K)
        il_col = jnp.transpose(il_vec, (1, 0))
        mil_orig = jnp.float32(0.0)
        mil_supp = jnp.float32(0.0)
        for i in range(_B):
            ge, je = i // 5, i % 5
            il_el = il_col[128 * ge + _RP * je:128 * ge + _RP * je + _C + 1]
            gs, js = (_B + i) // 5, (_B + i) % 5
            il_sp = il_col[128 * gs + _RP * js:128 * gs + _RP * js + _C + 1]
            mil_orig += _mil(il_el, labb_ref[i])
            mil_supp += _mil(il_sp, labs_ref[i])
        milv_ref[...] = jnp.concatenate(
            [jnp.reshape(mil_orig, (1, 1)), jnp.reshape(mil_supp, (1, 1)),
             jnp.zeros((1, 6), jnp.float32)], axis=1)


def kernel(feat, cas, attn, mask, v_atn, f_atn, labels):
    f32 = jnp.float32
    cas_t = jnp.transpose(cas, (0, 2, 1))
    atn_t = jnp.transpose(attn, (0, 2, 1))
    mask_t = jnp.transpose(mask, (0, 2, 1))
    v_t = jnp.transpose(v_atn, (0, 2, 1))
    f_t = jnp.transpose(f_atn, (0, 2, 1))
    labb = jnp.concatenate([labels, jnp.ones_like(labels[:, :1])], axis=1)[:, :, None]
    labs = jnp.concatenate([labels, jnp.zeros_like(labels[:, :1])], axis=1)[:, :, None]

    full = lambda shape: pl.BlockSpec(shape, lambda p, d: (0,) * len(shape))
    scal, milv, pairs = pl.pallas_call(
        _fused_kernel,
        grid=(_NPAIR, _ND),
        in_specs=[
            full((_B, _C + 1, _T)),
            full((_B, 1, _T)),
            full((_B, 1, _T)),
            full((_B, 1, _T)),
            full((_B, 1, _T)),
            full((_B, _C + 1, 1)),
            full((_B, _C + 1, 1)),
            full((_B, _T, _C + 1)),
            full((_B, _T, 1)),
            pl.BlockSpec((1, _DT, _D), lambda p, d: (2 * p, d, 0)),
            pl.BlockSpec((1, _DT, _D), lambda p, d: (2 * p + 1, d, 0)),
        ],
        out_specs=(pl.BlockSpec((1, 16), lambda p, d: (0, 0)),
                   pl.BlockSpec((1, 8), lambda p, d: (0, 0)),
                   pl.BlockSpec((1, 1, 2), lambda p, d: (p, 0, 0))),
        out_shape=(jax.ShapeDtypeStruct((1, 16), f32),
                   jax.ShapeDtypeStruct((1, 8), f32),
                   jax.ShapeDtypeStruct((_NPAIR, 1, 2), f32)),
        scratch_shapes=[pltpu.VMEM((2 * _B * _RP, _T), f32),
                        pltpu.VMEM((_T, 512), f32),
                        pltpu.VMEM((_T, 512), jnp.int32),
                        pltpu.VMEM((1, 512), jnp.int32),
                        pltpu.VMEM((2, _T, 48), f32),
                        pltpu.VMEM((2, 48, _D), f32)],
    )(cas_t, atn_t, mask_t, v_t, f_t, labb, labs, cas, attn, feat, feat)

    loss_contrastive = jnp.sum(pairs[:, 0, 0]) / jnp.sum(pairs[:, 0, 1])
    s = scal[0]
    inv = f32(0.1)
    mil_orig = milv[0, 0] * inv
    mil_supp = milv[0, 1] * inv
    mutual = s[0] * inv
    norm_avg = (s[1] + s[2] + s[3]) * (inv / 3.0)
    guide_avg = (s[4] + s[5] + s[6]) * (inv / 3.0)
    total = (mil_orig + mil_supp + loss_contrastive + mutual
             + 0.8 * norm_avg + 0.8 * guide_avg)
    return (total, mil_orig, mil_supp, loss_contrastive, mutual,
            norm_avg, guide_avg)


# speedup vs baseline: 1.3708x; 1.3708x over previous
import jax
import jax.numpy as jnp
from jax.experimental import pallas as pl
from jax.experimental.pallas import tpu as pltpu

_B, _T, _D, _C = 10, 2048, 2048, 20
_K = 292
_DT = 512
_ND = _D // _DT
_NPAIR = 3
_RP = 24
_INT_MIN = -(2 ** 31)
_M31 = 2 ** 31 - 1

_PASS_SCHED = {s: list(range(30 - 3 * (s - 1), 30 - 3 * s, -1))
               for s in range(1, 10)}
_PASS_SCHED[10] = [3, 2, 1, 0]


def _sortable(bits):
    return jnp.where(bits < 0, bits ^ _M31, bits)


def _mil(il, lab):
    lwb = lab * (1.0 / (jnp.sum(lab) + 1e-4))
    mx = jnp.max(il)
    ls = il - mx - jnp.log(jnp.sum(jnp.exp(il - mx)))
    return -jnp.sum(lwb * ls)


def _fused_kernel(cas_ref, atn_ref, mask_ref, v_ref, f_ref, labb_ref,
                  labs_ref, x1_ref, x2_ref, scal_ref, milv_ref, pairs_ref,
                  stage_ref, valt_ref, keyt_ref, t_ref, am_ref, acc_ref):
    p = pl.program_id(0)
    d = pl.program_id(1)
    sidx = p * _ND + d
    kf = jnp.float32(_K)

    def cnt_ge(c):
        parts = []
        for j in range(16):
            blk = keyt_ref[pl.ds(128 * j, 128), :]
            parts.append(jnp.sum((blk >= c).astype(jnp.float32),
                                 axis=0, keepdims=True))
        while len(parts) > 1:
            parts = [a + b for a, b in zip(parts[0::2], parts[1::2])]
        return parts[0]

    @pl.when(sidx == 0)
    def _build():
        mutual = jnp.float32(0.0)
        norm_a = jnp.float32(0.0)
        norm_v = jnp.float32(0.0)
        norm_f = jnp.float32(0.0)
        guide_a = jnp.float32(0.0)
        guide_v = jnp.float32(0.0)
        guide_f = jnp.float32(0.0)
        inv_t = jnp.float32(1.0 / _T)
        pad = jnp.full((_RP - _C - 1, _T), -jnp.inf, jnp.float32)
        for i in range(_B):
            cas = cas_ref[i]
            atn = atn_ref[i]
            msk = mask_ref[i]
            v = v_ref[i] * msk
            f = f_ref[i] * msk
            el = cas * msk
            atn_m = atn * msk
            mutual += jnp.mean((v - f) ** 2)
            mn = jnp.min(el, axis=0, keepdims=True)
            supp = atn_m * (el - mn) + mn
            stage_ref[pl.ds(_RP * i, _RP), :] = \
                jnp.concatenate([el, pad], axis=0)
            stage_ref[pl.ds(_RP * (_B + i), _RP), :] = \
                jnp.concatenate([supp, pad], axis=0)
            mx = jnp.max(el, axis=0, keepdims=True)
            z = jnp.sum(jnp.exp(el - mx), axis=0, keepdims=True)
            bg = jnp.exp(el[_C:_C + 1, :] - mx) / z
            norm_a += jnp.sum(atn_m) * inv_t
            norm_v += jnp.sum(v) * inv_t
            norm_f += jnp.sum(f) * inv_t
            guide_a += jnp.sum(jnp.abs(1.0 - atn_m - bg)) * inv_t
            guide_v += jnp.sum(jnp.abs(1.0 - v - bg)) * inv_t
            guide_f += jnp.sum(jnp.abs(1.0 - f - bg)) * inv_t
        ipad = jnp.full((8, _T), -jnp.inf, jnp.float32)
        for g in range(4):
            blk = jnp.concatenate(
                [stage_ref[pl.ds(120 * g, 120), :], ipad], axis=0)
            valt_ref[:, 128 * g:128 * (g + 1)] = jnp.transpose(blk, (1, 0))
        keyt_ref[...] = _sortable(
            jax.lax.bitcast_convert_type(valt_ref[...], jnp.int32))
        zero = jnp.zeros((1, 512), jnp.int32)
        t_ref[...] = jnp.where(cnt_ge(zero) >= kf, zero,
                               jnp.full((1, 512), _INT_MIN, jnp.int32))
        scal_ref[...] = jnp.concatenate(
            [jnp.reshape(s, (1, 1)) for s in
             (mutual, norm_a, norm_v, norm_f, guide_a, guide_v, guide_f)]
            + [jnp.zeros((1, 9), jnp.float32)], axis=1)

    for s, bits in _PASS_SCHED.items():
        @pl.when(sidx == s)
        def _passes(bits=bits):
            for b in bits:
                t = t_ref[...]
                cand = t + jnp.int32(1 << b)
                t_ref[...] = jnp.where(cnt_ge(cand) >= kf, cand, t)

    @pl.when(d == 0)
    def _amats():
        acc_ref[...] = jnp.zeros_like(acc_ref)
        for q in range(2):
            idx = 2 * p + q
            cas = cas_ref[idx]
            atn = atn_ref[idx]
            mnu = jnp.min(cas, axis=0, keepdims=True)
            st = atn * (cas - mnu) + mnu
            mxt = jnp.max(st, axis=1, keepdims=True)
            e = jnp.exp(st - mxt)
            zt = jnp.sum(e, axis=1, keepdims=True)
            a1 = e / zt
            al = (1.0 - a1) * jnp.float32(1.0 / (_T - 1))
            zpad = jnp.zeros((3, _T), jnp.float32)
            am_ref[q] = jnp.concatenate([a1, zpad, al, zpad], axis=0)

    dn = (((1,), (0,)), ((), ()))
    m1 = jax.lax.dot_general(am_ref[0], x1_ref[0], dn,
                             preferred_element_type=jnp.float32,
                             precision=jax.lax.Precision.DEFAULT)
    m2 = jax.lax.dot_general(am_ref[1], x2_ref[0], dn,
                             preferred_element_type=jnp.float32,
                             precision=jax.lax.Precision.DEFAULT)
    h1 = m1[0:_C + 1]
    l1 = m1[24:24 + _C + 1]
    h2 = m2[0:_C + 1]
    l2 = m2[24:24 + _C + 1]
    acc_ref[0] = acc_ref[0] + h1 * h2
    acc_ref[1] = acc_ref[1] + h1 * l2
    acc_ref[2] = acc_ref[2] + h2 * l1
    acc_ref[3] = acc_ref[3] + h1 * h1
    acc_ref[4] = acc_ref[4] + h2 * h2
    acc_ref[5] = acc_ref[5] + l1 * l1
    acc_ref[6] = acc_ref[6] + l2 * l2

    @pl.when(d == _ND - 1)
    def _pair_final():
        h1h2 = jnp.sum(acc_ref[0], axis=1, keepdims=True)
        h1l2 = jnp.sum(acc_ref[1], axis=1, keepdims=True)
        h2l1 = jnp.sum(acc_ref[2], axis=1, keepdims=True)
        nh1 = jnp.sqrt(jnp.sum(acc_ref[3], axis=1, keepdims=True))
        nh2 = jnp.sqrt(jnp.sum(acc_ref[4], axis=1, keepdims=True))
        nl1 = jnp.sqrt(jnp.sum(acc_ref[5], axis=1, keepdims=True))
        nl2 = jnp.sqrt(jnp.sum(acc_ref[6], axis=1, keepdims=True))
        d1 = 1.0 - h1h2 / (nh1 * nh2)
        d2 = 1.0 - h1l2 / (nh1 * nl2)
        d3 = 1.0 - h2l1 / (nh2 * nl1)
        ll = labs_ref[2 * p] * labs_ref[2 * p + 1]
        part = 0.5 * (jnp.sum(jnp.maximum(d1 - d2 + 0.5, 0.0) * ll)
                      + jnp.sum(jnp.maximum(d1 - d3 + 0.5, 0.0) * ll))
        ntmp = jnp.sum(ll)
        pairs_ref[0] = jnp.concatenate(
            [jnp.reshape(part, (1, 1)), jnp.reshape(ntmp, (1, 1))], axis=1)

    @pl.when(sidx == _NPAIR * _ND - 1)
    def _finalize():
        t = t_ref[...]
        cparts = []
        sparts = []
        for j in range(16):
            kblk = keyt_ref[pl.ds(128 * j, 128), :]
            vblk = valt_ref[pl.ds(128 * j, 128), :]
            gtb = kblk > t
            cparts.append(jnp.sum(gtb.astype(jnp.float32),
                                  axis=0, keepdims=True))
            sparts.append(jnp.sum(jnp.where(gtb, vblk, jnp.float32(0.0)),
                                  axis=0, keepdims=True))
        while len(cparts) > 1:
            cparts = [a + b for a, b in zip(cparts[0::2], cparts[1::2])]
            sparts = [a + b for a, b in zip(sparts[0::2], sparts[1::2])]
        tval = jax.lax.bitcast_convert_type(_sortable(t), jnp.float32)
        il_vec = (sparts[0] + (kf - cparts[0]) * tval) \
            * jnp.float32(1.0 / _K)
        il_col = jnp.transpose(il_vec, (1, 0))
        mil_orig = jnp.float32(0.0)
        mil_supp = jnp.float32(0.0)
        for i in range(_B):
            ge, je = i // 5, i % 5
            il_el = il_col[128 * ge + _RP * je:128 * ge + _RP * je + _C + 1]
            gs, js = (_B + i) // 5, (_B + i) % 5
            il_sp = il_col[128 * gs + _RP * js:128 * gs + _RP * js + _C + 1]
            mil_orig += _mil(il_el, labb_ref[i])
            mil_supp += _mil(il_sp, labs_ref[i])
        milv_ref[...] = jnp.concatenate(
            [jnp.reshape(mil_orig, (1, 1)), jnp.reshape(mil_supp, (1, 1)),
             jnp.zeros((1, 6), jnp.float32)], axis=1)


def kernel(feat, cas, attn, mask, v_atn, f_atn, labels):
    f32 = jnp.float32
    cas_t = jnp.transpose(cas, (0, 2, 1))
    atn_t = jnp.transpose(attn, (0, 2, 1))
    mask_t = jnp.transpose(mask, (0, 2, 1))
    v_t = jnp.transpose(v_atn, (0, 2, 1))
    f_t = jnp.transpose(f_atn, (0, 2, 1))
    labb = jnp.concatenate([labels, jnp.ones_like(labels[:, :1])], axis=1)[:, :, None]
    labs = jnp.concatenate([labels, jnp.zeros_like(labels[:, :1])], axis=1)[:, :, None]

    full = lambda shape: pl.BlockSpec(shape, lambda p, d: (0,) * len(shape))
    scal, milv, pairs = pl.pallas_call(
        _fused_kernel,
        grid=(_NPAIR, _ND),
        in_specs=[
            full((_B, _C + 1, _T)),
            full((_B, 1, _T)),
            full((_B, 1, _T)),
            full((_B, 1, _T)),
            full((_B, 1, _T)),
            full((_B, _C + 1, 1)),
            full((_B, _C + 1, 1)),
            pl.BlockSpec((1, _T, _DT), lambda p, d: (2 * p, 0, d)),
            pl.BlockSpec((1, _T, _DT), lambda p, d: (2 * p + 1, 0, d)),
        ],
        out_specs=(pl.BlockSpec((1, 16), lambda p, d: (0, 0)),
                   pl.BlockSpec((1, 8), lambda p, d: (0, 0)),
                   pl.BlockSpec((1, 1, 2), lambda p, d: (p, 0, 0))),
        out_shape=(jax.ShapeDtypeStruct((1, 16), f32),
                   jax.ShapeDtypeStruct((1, 8), f32),
                   jax.ShapeDtypeStruct((_NPAIR, 1, 2), f32)),
        scratch_shapes=[pltpu.VMEM((2 * _B * _RP, _T), f32),
                        pltpu.VMEM((_T, 512), f32),
                        pltpu.VMEM((_T, 512), jnp.int32),
                        pltpu.VMEM((1, 512), jnp.int32),
                        pltpu.VMEM((2, 48, _T), f32),
                        pltpu.VMEM((7, _C + 1, _DT), f32)],
    )(cas_t, atn_t, mask_t, v_t, f_t, labb, labs, feat, feat)

    loss_contrastive = jnp.sum(pairs[:, 0, 0]) / jnp.sum(pairs[:, 0, 1])
    s = scal[0]
    inv = f32(0.1)
    mil_orig = milv[0, 0] * inv
    mil_supp = milv[0, 1] * inv
    mutual = s[0] * inv
    norm_avg = (s[1] + s[2] + s[3]) * (inv / 3.0)
    guide_avg = (s[4] + s[5] + s[6]) * (inv / 3.0)
    total = (mil_orig + mil_supp + loss_contrastive + mutual
             + 0.8 * norm_avg + 0.8 * guide_avg)
    return (total, mil_orig, mil_supp, loss_contrastive, mutual,
            norm_avg, guide_avg)
